# Initial kernel scaffold; baseline (speedup 1.0000x reference)
import jax
import jax.numpy as jnp
from jax.experimental import pallas as pl


def kernel(x, embed_table, W, b):
    raise NotImplementedError("write your pallas kernel here")


# speedup vs baseline: 25.1045x; 25.1045x over previous
import functools

import jax
import jax.numpy as jnp
from jax import lax
from jax.experimental import pallas as pl
from jax.experimental.pallas import tpu as pltpu
from jax.experimental.pallas import tpu_sc as plsc

VOCAB = 100000
EMBED_DIM = 128
NUM_CLASSES = 2
BATCH_N = 4096
SEQ_LEN = 200

BK = 2048
N_BLOCKS = 49
VOCAB_PAD = BK * N_BLOCKS
N_TILES = 16
ROWS_PER_TILE = BATCH_N // N_TILES
GROUPS = ROWS_PER_TILE // 16


def _tc_project(w_ref, t_ref, o_ref):
    o_ref[...] = lax.dot_general(
        w_ref[...], t_ref[...],
        dimension_numbers=(((0,), (1,)), ((), ())),
        preferred_element_type=jnp.float32,
    ) * (1.0 / SEQ_LEN)


def _sc_pool(pt_hbm, x_hbm, b_hbm, out_hbm, p_v, x_v, res_v, b_v):
    cls = lax.axis_index("c")
    tile = lax.axis_index("s")
    base = tile * ROWS_PER_TILE
    pltpu.sync_copy(pt_hbm.at[cls], p_v)
    pltpu.sync_copy(b_hbm.at[cls], b_v)
    lanes = lax.iota(jnp.int32, 16)

    def group_body(g, carry):
        pltpu.sync_copy(x_hbm.at[pl.ds((base + g * 16) * SEQ_LEN, 16 * SEQ_LEN)],
                        x_v)
        row_off = lanes * SEQ_LEN

        def seq_body(s, acc):
            xv = plsc.load_gather(x_v, [row_off + s])
            return acc + plsc.load_gather(p_v, [xv])

        res_v[pl.ds(g * 16, 16)] = lax.fori_loop(0, SEQ_LEN, seq_body, b_v[...])
        return carry

    lax.fori_loop(0, GROUPS, group_body, 0)
    pltpu.sync_copy(res_v, out_hbm.at[cls, pl.ds(base, ROWS_PER_TILE)])


def kernel(x, embed_table, W, b):
    pt = pl.pallas_call(
        _tc_project,
        grid=(N_BLOCKS,),
        in_specs=[
            pl.BlockSpec((EMBED_DIM, NUM_CLASSES), lambda i: (0, 0)),
            pl.BlockSpec((BK, EMBED_DIM), lambda i: (i, 0)),
        ],
        out_specs=pl.BlockSpec((NUM_CLASSES, BK), lambda i: (0, i)),
        out_shape=jax.ShapeDtypeStruct((NUM_CLASSES, VOCAB_PAD), jnp.float32),
    )(W, embed_table)

    b16 = jnp.broadcast_to(b[:, None], (NUM_CLASSES, 16))

    sc_fn = pl.kernel(
        _sc_pool,
        mesh=plsc.VectorSubcoreMesh(core_axis_name="c", subcore_axis_name="s"),
        out_type=jax.ShapeDtypeStruct((NUM_CLASSES, BATCH_N), jnp.float32),
        scratch_types=[
            pltpu.VMEM((VOCAB_PAD,), jnp.float32),
            pltpu.VMEM((16 * SEQ_LEN,), jnp.int32),
            pltpu.VMEM((ROWS_PER_TILE,), jnp.float32),
            pltpu.VMEM((16,), jnp.float32),
        ],
        compiler_params=pltpu.CompilerParams(needs_layout_passes=False),
    )
    out_t = sc_fn(pt, x.reshape(-1), b16)
    return out_t.T
